# Initial kernel scaffold; baseline (speedup 1.0000x reference)
import jax
import jax.numpy as jnp
from jax.experimental import pallas as pl


def kernel(x, pos_table):
    raise NotImplementedError("write your pallas kernel here")


# speedup vs baseline: 2.8358x; 2.8358x over previous
import jax
import jax.numpy as jnp
from jax.experimental import pallas as pl

BATCH = 4
SEQ_LEN = 4096
EMBED_DIM = 1024
SEQ_BLOCK = 512


def _add_block(x_ref, pos_ref, o_ref):
    o_ref[...] = x_ref[...] + pos_ref[...]


def kernel(x, pos_table):
    batch, seq_len, embed_dim = x.shape
    n_seq = seq_len // SEQ_BLOCK
    pos = pos_table[:seq_len]
    return pl.pallas_call(
        _add_block,
        grid=(n_seq, batch),
        in_specs=[
            pl.BlockSpec((1, SEQ_BLOCK, embed_dim), lambda i, j: (j, i, 0)),
            pl.BlockSpec((SEQ_BLOCK, embed_dim), lambda i, j: (i, 0)),
        ],
        out_specs=pl.BlockSpec((1, SEQ_BLOCK, embed_dim), lambda i, j: (j, i, 0)),
        out_shape=jax.ShapeDtypeStruct((batch, seq_len, embed_dim), x.dtype),
    )(x, pos)


# speedup vs baseline: 3.2964x; 1.1624x over previous
import jax
import jax.numpy as jnp
from jax.experimental import pallas as pl

BATCH = 4
SEQ_LEN = 4096
EMBED_DIM = 1024
SEQ_BLOCK = 512


def _add_block(x_ref, pos_ref, o_ref):
    o_ref[...] = x_ref[...] + pos_ref[...]


def kernel(x, pos_table):
    batch, seq_len, embed_dim = x.shape
    n_seq = seq_len // SEQ_BLOCK
    pos = pos_table[:seq_len]
    return pl.pallas_call(
        _add_block,
        grid=(n_seq,),
        in_specs=[
            pl.BlockSpec((batch, SEQ_BLOCK, embed_dim), lambda i: (0, i, 0)),
            pl.BlockSpec((SEQ_BLOCK, embed_dim), lambda i: (i, 0)),
        ],
        out_specs=pl.BlockSpec((batch, SEQ_BLOCK, embed_dim), lambda i: (0, i, 0)),
        out_shape=jax.ShapeDtypeStruct((batch, seq_len, embed_dim), x.dtype),
    )(x, pos)
